# Initial kernel scaffold; baseline (speedup 1.0000x reference)
import jax
import jax.numpy as jnp
from jax.experimental import pallas as pl


def kernel(node_feat, edge_feat, edge_dst, node_graph_indices, virtual_node_emb, W1, b1, ln_g, ln_b, W2, b2, We, be):
    raise NotImplementedError("write your pallas kernel here")


# speedup vs baseline: 4.1124x; 4.1124x over previous
import functools
import math

import jax
import jax.numpy as jnp
from jax import lax
from jax.experimental import pallas as pl
from jax.experimental.pallas import tpu as pltpu
from jax.experimental.pallas import tpu_sc as plsc

N = 10000
E = 320000
D = 128
B = 100

NC = 2
NS = 16
EBLK = 128
NBLK = E // EBLK
LH = D // NC
BPT = NBLK // NS
XTRA = NBLK - BPT * NS
NGRP = (BPT + 1 + 7) // 8
DST_PAD_ROWS = max(0, BPT * (NS - 1) + min(NS - 1, XTRA) + 8 * NGRP - NBLK)
RPT_A = N // NS
_ZCHUNKS_A = (128, 128, 128, 128, 113)
DEGW = 16
RPT_D = 640
NPAD_D = NS * RPT_D

_HIGH = lax.Precision.HIGHEST


def _sc_segsum_body(edge_feat_hbm, edge_dst_hbm, acc_out, deg_out,
                    idx_buf, feat_buf, ones_buf, acc_sp, deg_sp):
    c = lax.axis_index("c")
    s = lax.axis_index("s")

    def _fill(i, _):
        ones_buf[i, :] = jnp.zeros((DEGW,), jnp.float32)
        for j in range(LH // 16):
            feat_buf[i, pl.ds(j * 16, 16)] = jnp.zeros((16,), jnp.float32)
        return 0
    lax.fori_loop(0, EBLK, _fill, 0)

    abase = s * RPT_A
    off = 0
    for zc in _ZCHUNKS_A:
        pltpu.sync_copy(feat_buf.at[pl.ds(0, zc)],
                        acc_sp.at[pl.ds(abase + off, zc)])
        off += zc
    dbase = s * RPT_D

    @pl.when(c == 0)
    def _():
        for r in range(RPT_D // EBLK):
            pltpu.sync_copy(ones_buf, deg_sp.at[pl.ds(dbase + r * EBLK,
                                                      EBLK)])

    def _refill(i, _):
        ones_buf[i, :] = jnp.ones((DEGW,), jnp.float32)
        return 0
    lax.fori_loop(0, EBLK, _refill, 0)

    plsc.subcore_barrier()

    s0 = s * BPT + jnp.minimum(s, XTRA)
    nblocks = BPT + (s < XTRA).astype(jnp.int32)

    def _grp(g, _):
        pltpu.sync_copy(edge_dst_hbm.at[pl.ds(s0 + g * 8, 8)], idx_buf)

        def _step(jj, _):
            b = s0 + g * 8 + jj
            pltpu.sync_copy(
                edge_feat_hbm.at[pl.ds(b * EBLK, EBLK), pl.ds(c * LH, LH)],
                feat_buf)
            pltpu.sync_copy(feat_buf, acc_sp.at[idx_buf.at[jj]], add=True)

            @pl.when(c == 0)
            def _():
                pltpu.sync_copy(ones_buf, deg_sp.at[idx_buf.at[jj]],
                                add=True)
            return 0
        lax.fori_loop(0, jnp.minimum(8, nblocks - g * 8), _step, 0)
        return 0
    lax.fori_loop(0, NGRP, _grp, 0)

    plsc.subcore_barrier()

    pltpu.sync_copy(acc_sp.at[pl.ds(abase, RPT_A)],
                    acc_out.at[pl.ds(abase, RPT_A), pl.ds(c * LH, LH)])

    @pl.when(c == 0)
    def _():
        pltpu.sync_copy(deg_sp.at[pl.ds(dbase, RPT_D)],
                        deg_out.at[pl.ds(dbase, RPT_D)])


@functools.cache
def _sc_segsum():
    return pl.kernel(
        _sc_segsum_body,
        out_type=[jax.ShapeDtypeStruct((N, D), jnp.float32),
                  jax.ShapeDtypeStruct((NPAD_D, DEGW), jnp.float32)],
        mesh=plsc.VectorSubcoreMesh(core_axis_name="c", subcore_axis_name="s",
                                    num_cores=NC, num_subcores=NS),
        scratch_types=[
            pltpu.VMEM((8, EBLK), jnp.int32),
            pltpu.VMEM((EBLK, LH), jnp.float32),
            pltpu.VMEM((EBLK, DEGW), jnp.float32),
            pltpu.VMEM_SHARED((N, LH), jnp.float32),
            pltpu.VMEM_SHARED((NPAD_D, DEGW), jnp.float32),
        ],
        compiler_params=pltpu.CompilerParams(use_tc_tiling_on_sc=False),
    )


NROWS = 1000
NGRID = N // NROWS


def _vn_body(ids_ref, nf_ref, vne_ref, W1_ref, b1_ref, lng_ref, lnb_ref,
             W2_ref, b2_ref, vn_ref, acc_ref):
    i = pl.program_id(0)

    @pl.when(i == 0)
    def _():
        acc_ref[...] = jnp.zeros_like(acc_ref)

    ids = ids_ref[0, 0, :]
    onehot = (ids[:, None] ==
              lax.broadcasted_iota(jnp.int32, (NROWS, B), 1)).astype(jnp.float32)
    aug = jnp.concatenate(
        [nf_ref[...], jnp.ones((NROWS, 1), jnp.float32)], axis=1)
    acc_ref[...] += lax.dot_general(onehot, aug, (((0,), (0,)), ((), ())),
                                    precision=_HIGH)

    @pl.when(i == NGRID - 1)
    def _():
        sums = acc_ref[:, 0:D]
        cnt = acc_ref[:, D:D + 1]
        gmean = sums / jnp.maximum(cnt, 1.0)
        vne_h = lax.dot_general(vne_ref[...], W1_ref[0:D, :],
                                (((1,), (0,)), ((), ())), precision=_HIGH)
        h = vne_h + b1_ref[...] + lax.dot_general(
            gmean, W1_ref[D:2 * D, :], (((1,), (0,)), ((), ())),
            precision=_HIGH)
        h = 0.5 * h * (1.0 + lax.erf(h * (1.0 / math.sqrt(2.0))))
        mu = jnp.mean(h, axis=1, keepdims=True)
        var = jnp.mean((h - mu) ** 2, axis=1, keepdims=True)
        h = (h - mu) * lax.rsqrt(var + 1e-5) * lng_ref[...] + lnb_ref[...]
        vn_ref[...] = b2_ref[...] + lax.dot_general(
            h, W2_ref[...], (((1,), (0,)), ((), ())), precision=_HIGH)


def _vn_call(ids3, node_feat, vne, W1, b1r, lngr, lnbr, W2, b2r):
    return pl.pallas_call(
        _vn_body,
        grid=(NGRID,),
        in_specs=[
            pl.BlockSpec((1, 1, NROWS), lambda i: (i, 0, 0)),
            pl.BlockSpec((NROWS, D), lambda i: (i, 0)),
            pl.BlockSpec((1, D), lambda i: (0, 0)),
            pl.BlockSpec((2 * D, D), lambda i: (0, 0)),
            pl.BlockSpec((1, D), lambda i: (0, 0)),
            pl.BlockSpec((1, D), lambda i: (0, 0)),
            pl.BlockSpec((1, D), lambda i: (0, 0)),
            pl.BlockSpec((D, D), lambda i: (0, 0)),
            pl.BlockSpec((1, D), lambda i: (0, 0)),
        ],
        out_specs=pl.BlockSpec((B, D), lambda i: (0, 0)),
        out_shape=jax.ShapeDtypeStruct((B, D), jnp.float32),
        scratch_shapes=[pltpu.VMEM((B, D + 1), jnp.float32)],
    )(ids3, node_feat, vne, W1, b1r, lngr, lnbr, W2, b2r)


def _nf_body(ids_ref, nf_ref, acc_ref, deg_ref, vn_ref, We_ref, be_ref,
             out_ref):
    ids = ids_ref[0, 0, :]
    onehot = (ids[:, None] ==
              lax.broadcasted_iota(jnp.int32, (NROWS, B), 1)).astype(jnp.float32)
    vnb = lax.dot_general(onehot, vn_ref[...], (((1,), (0,)), ((), ())),
                          precision=_HIGH)
    asum = acc_ref[...]
    deg = deg_ref[0, 0, :][:, None]
    emean = asum / jnp.maximum(deg, 1.0)
    e = lax.dot_general(emean, We_ref[...], (((1,), (0,)), ((), ())),
                        precision=_HIGH)
    e = e + be_ref[...] * (deg > 0.0).astype(jnp.float32)
    out_ref[...] = nf_ref[...] + vnb + e


def _nf_call(ids3, node_feat, acc, deg3, vn, We, ber):
    return pl.pallas_call(
        _nf_body,
        grid=(NGRID,),
        in_specs=[
            pl.BlockSpec((1, 1, NROWS), lambda i: (i, 0, 0)),
            pl.BlockSpec((NROWS, D), lambda i: (i, 0)),
            pl.BlockSpec((NROWS, D), lambda i: (i, 0)),
            pl.BlockSpec((1, 1, NROWS), lambda i: (i, 0, 0)),
            pl.BlockSpec((B, D), lambda i: (0, 0)),
            pl.BlockSpec((D, D), lambda i: (0, 0)),
            pl.BlockSpec((1, D), lambda i: (0, 0)),
        ],
        out_specs=pl.BlockSpec((NROWS, D), lambda i: (i, 0)),
        out_shape=jax.ShapeDtypeStruct((N, D), jnp.float32),
    )(ids3, node_feat, acc, deg3, vn, We, ber)


def kernel(node_feat, edge_feat, edge_dst, node_graph_indices,
           virtual_node_emb, W1, b1, ln_g, ln_b, W2, b2, We, be):
    edge_dst2 = jnp.pad(edge_dst.reshape(NBLK, EBLK),
                        ((0, DST_PAD_ROWS), (0, 0)))
    ids3 = node_graph_indices.reshape(NGRID, 1, NROWS)

    acc, degacc = _sc_segsum()(edge_feat, edge_dst2)
    deg3 = degacc[:N, 0].reshape(NGRID, 1, NROWS)
    vn = _vn_call(ids3, node_feat, virtual_node_emb, W1,
                  b1.reshape(1, D), ln_g.reshape(1, D), ln_b.reshape(1, D),
                  W2, b2.reshape(1, D))
    nf = _nf_call(ids3, node_feat, acc, deg3, vn, We, be.reshape(1, D))
    return nf, vn


# speedup vs baseline: 6.7195x; 1.6339x over previous
import functools
import math

import jax
import jax.numpy as jnp
from jax import lax
from jax.experimental import pallas as pl
from jax.experimental.pallas import tpu as pltpu
from jax.experimental.pallas import tpu_sc as plsc

N = 10000
E = 320000
D = 128
B = 100

NC = 2
NS = 16
EBLK = 128
NBLK = E // EBLK
LH = D // NC
BPT = NBLK // NS
XTRA = NBLK - BPT * NS
NPAIR = BPT // 2
IDXROWS = BPT + 4
DST_PAD_ROWS = max(0, BPT * (NS - 1) + min(NS - 1, XTRA) + IDXROWS - NBLK)
RPT_A = N // NS
_ZCHUNKS_A = (128, 128, 128, 128, 113)
DEGW = 16
RPT_D = 640
NPAD_D = NS * RPT_D

_HIGH = lax.Precision.HIGHEST


def _sc_segsum_body(edge_feat_hbm, edge_dst_hbm, acc_out, deg_out,
                    idx_all, fbA, fbB, ones_buf, semA, semB, semO,
                    acc_sp, deg_sp):
    c = lax.axis_index("c")
    s = lax.axis_index("s")

    def _fill(i, _):
        ones_buf[i, :] = jnp.zeros((DEGW,), jnp.float32)
        for j in range(LH // 16):
            fbA[i, pl.ds(j * 16, 16)] = jnp.zeros((16,), jnp.float32)
        return 0
    lax.fori_loop(0, EBLK, _fill, 0)

    abase = s * RPT_A
    off = 0
    for zc in _ZCHUNKS_A:
        pltpu.sync_copy(fbA.at[pl.ds(0, zc)],
                        acc_sp.at[pl.ds(abase + off, zc)])
        off += zc
    dbase = s * RPT_D
    for r in range(RPT_D // EBLK):
        pltpu.sync_copy(ones_buf, deg_sp.at[pl.ds(dbase + r * EBLK, EBLK)])

    def _refill(i, _):
        ones_buf[i, :] = jnp.ones((DEGW,), jnp.float32)
        return 0
    lax.fori_loop(0, EBLK, _refill, 0)

    s0 = s * BPT + jnp.minimum(s, XTRA)
    pltpu.sync_copy(edge_dst_hbm.at[pl.ds(s0, IDXROWS)], idx_all)

    plsc.subcore_barrier()

    def _fetch(b, fb, sem):
        pltpu.async_copy(
            edge_feat_hbm.at[pl.ds(b * EBLK, EBLK), pl.ds(c * LH, LH)],
            fb, sem)

    def _waitf(fb, sem):
        pltpu.make_async_copy(
            edge_feat_hbm.at[pl.ds(0, EBLK), pl.ds(0, LH)], fb, sem).wait()

    def _half(j, fb, sem, deg_core):
        _waitf(fb, sem)

        @pl.when(c == deg_core)
        def _():
            pltpu.async_copy(ones_buf, deg_sp.at[idx_all.at[j]], semO,
                             add=True)
        pltpu.sync_copy(fb, acc_sp.at[idx_all.at[j]], add=True)

        @pl.when(c == deg_core)
        def _():
            pltpu.make_async_copy(ones_buf, deg_sp.at[idx_all.at[j]],
                                  semO).wait()

    _fetch(s0, fbA, semA)
    _fetch(s0 + 1, fbB, semB)

    def _pair(p, _):
        j0 = 2 * p
        _half(j0, fbA, semA, 0)

        @pl.when(p < NPAIR - 1)
        def _():
            _fetch(s0 + j0 + 2, fbA, semA)
        _half(j0 + 1, fbB, semB, 1)

        @pl.when(p < NPAIR - 1)
        def _():
            _fetch(s0 + j0 + 3, fbB, semB)
        return 0
    lax.fori_loop(0, NPAIR, _pair, 0)

    @pl.when(s < XTRA)
    def _():
        _fetch(s0 + BPT, fbA, semA)
        _half(BPT, fbA, semA, 0)

    plsc.subcore_barrier()

    pltpu.sync_copy(acc_sp.at[pl.ds(abase, RPT_A)],
                    acc_out.at[pl.ds(abase, RPT_A), pl.ds(c * LH, LH)])
    pltpu.sync_copy(deg_sp.at[pl.ds(dbase, RPT_D)],
                    deg_out.at[c, pl.ds(dbase, RPT_D)])


@functools.cache
def _sc_segsum():
    return pl.kernel(
        _sc_segsum_body,
        out_type=[jax.ShapeDtypeStruct((N, D), jnp.float32),
                  jax.ShapeDtypeStruct((NC, NPAD_D, DEGW), jnp.float32)],
        mesh=plsc.VectorSubcoreMesh(core_axis_name="c", subcore_axis_name="s",
                                    num_cores=NC, num_subcores=NS),
        scratch_types=[
            pltpu.VMEM((IDXROWS, EBLK), jnp.int32),
            pltpu.VMEM((EBLK, LH), jnp.float32),
            pltpu.VMEM((EBLK, LH), jnp.float32),
            pltpu.VMEM((EBLK, DEGW), jnp.float32),
            pltpu.SemaphoreType.DMA,
            pltpu.SemaphoreType.DMA,
            pltpu.SemaphoreType.DMA,
            pltpu.VMEM_SHARED((N, LH), jnp.float32),
            pltpu.VMEM_SHARED((NPAD_D, DEGW), jnp.float32),
        ],
        compiler_params=pltpu.CompilerParams(use_tc_tiling_on_sc=False),
    )


NROWS = 1000
NGRID = N // NROWS


def _vn_body(ids_ref, nf_ref, vne_ref, W1_ref, b1_ref, lng_ref, lnb_ref,
             W2_ref, b2_ref, vn_ref, acc_ref):
    i = pl.program_id(0)

    @pl.when(i == 0)
    def _():
        acc_ref[...] = jnp.zeros_like(acc_ref)

    ids = ids_ref[0, 0, :]
    onehot = (ids[:, None] ==
              lax.broadcasted_iota(jnp.int32, (NROWS, B), 1)).astype(jnp.float32)
    aug = jnp.concatenate(
        [nf_ref[...], jnp.ones((NROWS, 1), jnp.float32)], axis=1)
    acc_ref[...] += lax.dot_general(onehot, aug, (((0,), (0,)), ((), ())),
                                    precision=_HIGH)

    @pl.when(i == NGRID - 1)
    def _():
        sums = acc_ref[:, 0:D]
        cnt = acc_ref[:, D:D + 1]
        gmean = sums / jnp.maximum(cnt, 1.0)
        vne_h = lax.dot_general(vne_ref[...], W1_ref[0:D, :],
                                (((1,), (0,)), ((), ())), precision=_HIGH)
        h = vne_h + b1_ref[...] + lax.dot_general(
            gmean, W1_ref[D:2 * D, :], (((1,), (0,)), ((), ())),
            precision=_HIGH)
        h = 0.5 * h * (1.0 + lax.erf(h * (1.0 / math.sqrt(2.0))))
        mu = jnp.mean(h, axis=1, keepdims=True)
        var = jnp.mean((h - mu) ** 2, axis=1, keepdims=True)
        h = (h - mu) * lax.rsqrt(var + 1e-5) * lng_ref[...] + lnb_ref[...]
        vn_ref[...] = b2_ref[...] + lax.dot_general(
            h, W2_ref[...], (((1,), (0,)), ((), ())), precision=_HIGH)


def _vn_call(ids3, node_feat, vne, W1, b1r, lngr, lnbr, W2, b2r):
    return pl.pallas_call(
        _vn_body,
        grid=(NGRID,),
        in_specs=[
            pl.BlockSpec((1, 1, NROWS), lambda i: (i, 0, 0)),
            pl.BlockSpec((NROWS, D), lambda i: (i, 0)),
            pl.BlockSpec((1, D), lambda i: (0, 0)),
            pl.BlockSpec((2 * D, D), lambda i: (0, 0)),
            pl.BlockSpec((1, D), lambda i: (0, 0)),
            pl.BlockSpec((1, D), lambda i: (0, 0)),
            pl.BlockSpec((1, D), lambda i: (0, 0)),
            pl.BlockSpec((D, D), lambda i: (0, 0)),
            pl.BlockSpec((1, D), lambda i: (0, 0)),
        ],
        out_specs=pl.BlockSpec((B, D), lambda i: (0, 0)),
        out_shape=jax.ShapeDtypeStruct((B, D), jnp.float32),
        scratch_shapes=[pltpu.VMEM((B, D + 1), jnp.float32)],
    )(ids3, node_feat, vne, W1, b1r, lngr, lnbr, W2, b2r)


def _nf_body(ids_ref, nf_ref, acc_ref, deg_ref, vn_ref, We_ref, be_ref,
             out_ref):
    ids = ids_ref[0, 0, :]
    onehot = (ids[:, None] ==
              lax.broadcasted_iota(jnp.int32, (NROWS, B), 1)).astype(jnp.float32)
    vnb = lax.dot_general(onehot, vn_ref[...], (((1,), (0,)), ((), ())),
                          precision=_HIGH)
    asum = acc_ref[...]
    deg = deg_ref[0, 0, :][:, None]
    emean = asum / jnp.maximum(deg, 1.0)
    e = lax.dot_general(emean, We_ref[...], (((1,), (0,)), ((), ())),
                        precision=_HIGH)
    e = e + be_ref[...] * (deg > 0.0).astype(jnp.float32)
    out_ref[...] = nf_ref[...] + vnb + e


def _nf_call(ids3, node_feat, acc, deg3, vn, We, ber):
    return pl.pallas_call(
        _nf_body,
        grid=(NGRID,),
        in_specs=[
            pl.BlockSpec((1, 1, NROWS), lambda i: (i, 0, 0)),
            pl.BlockSpec((NROWS, D), lambda i: (i, 0)),
            pl.BlockSpec((NROWS, D), lambda i: (i, 0)),
            pl.BlockSpec((1, 1, NROWS), lambda i: (i, 0, 0)),
            pl.BlockSpec((B, D), lambda i: (0, 0)),
            pl.BlockSpec((D, D), lambda i: (0, 0)),
            pl.BlockSpec((1, D), lambda i: (0, 0)),
        ],
        out_specs=pl.BlockSpec((NROWS, D), lambda i: (i, 0)),
        out_shape=jax.ShapeDtypeStruct((N, D), jnp.float32),
    )(ids3, node_feat, acc, deg3, vn, We, ber)


def kernel(node_feat, edge_feat, edge_dst, node_graph_indices,
           virtual_node_emb, W1, b1, ln_g, ln_b, W2, b2, We, be):
    edge_dst2 = jnp.pad(edge_dst.reshape(NBLK, EBLK),
                        ((0, DST_PAD_ROWS), (0, 0)))
    ids3 = node_graph_indices.reshape(NGRID, 1, NROWS)

    acc, degacc = _sc_segsum()(edge_feat, edge_dst2)
    deg3 = (degacc[0, :N, 0] + degacc[1, :N, 0]).reshape(NGRID, 1, NROWS)
    vn = _vn_call(ids3, node_feat, virtual_node_emb, W1,
                  b1.reshape(1, D), ln_g.reshape(1, D), ln_b.reshape(1, D),
                  W2, b2.reshape(1, D))
    nf = _nf_call(ids3, node_feat, acc, deg3, vn, We, be.reshape(1, D))
    return nf, vn
